# baseline (speedup 1.0000x reference)
import functools

import jax
import jax.numpy as jnp
from jax import lax
from jax.experimental import pallas as pl
from jax.experimental.pallas import tpu as pltpu
from jax.experimental.pallas import tpu_sc as plsc

_N = 32768
_B = 128
_K = 3277
_BOOST = 1.0

_NC, _NS, _L = 2, 16, 16
_NW = _NC * _NS
_RPW = _B // _NW
_NBINS = 256
_NLEV = 4


def _encode(bits):
    m = lax.shift_right_arithmetic(bits, 31)
    return bits ^ (m | jnp.int32(-2147483648))


def _srl(v, amount):
    return lax.shift_right_logical(v, lax.full_like(v, amount))


def _sc_body(x_hbm, scale_hbm, thr_hbm, xrow, scalev, hist, thrv):
    cid = lax.axis_index("c")
    sid = lax.axis_index("s")
    wid = sid * _NC + cid
    pltpu.sync_copy(scale_hbm, scalev)
    lanes = lax.iota(jnp.int32, _L)
    lane_base = lanes * _NBINS
    ones = jnp.ones((_L,), jnp.int32)

    def row_body(rl, thr_vec):
        pltpu.sync_copy(x_hbm.at[wid * _RPW + rl], xrow)
        pref = jnp.int32(0)
        kr = jnp.int32(_K)
        for lev in range(_NLEV):
            shift = 24 - 8 * lev

            def zero_body(j, _):
                hist[pl.ds(j * _L, _L)] = jnp.zeros((_L,), jnp.int32)
                return 0

            lax.fori_loop(0, (_NBINS * _L) // _L, zero_body, 0)

            def scan_body(i, _, lev=lev, shift=shift, pref=pref):
                xv = xrow[pl.ds(i * _L, _L)]
                sv = scalev[pl.ds(i * _L, _L)]
                u = _encode(lax.bitcast_convert_type(xv * sv, jnp.int32))
                binv = _srl(u, shift) & 255
                idx = lane_base + binv
                if lev == 0:
                    plsc.addupdate_scatter(hist, [idx], ones)
                else:
                    match = _srl(u, shift + 8) == pref
                    plsc.addupdate_scatter(hist, [idx], ones, mask=match)
                return 0

            lax.fori_loop(0, _N // _L, scan_body, 0)

            def chunk_body(j, carry):
                csum, cnt_ok, sum_sat = carry
                cbase = (_NBINS // _L - 1 - j) * _L

                def lane_red(l, acc):
                    return acc + hist[pl.ds(l * _NBINS + cbase, _L)]

                counts = lax.fori_loop(0, _L, lane_red,
                                       jnp.zeros((_L,), jnp.int32))
                rc = lax.rev(counts, (0,))
                rs = jnp.cumsum(rc) + csum
                sat = (rs >= kr).astype(jnp.int32)
                return (csum + jnp.sum(rc), cnt_ok + jnp.sum(sat),
                        sum_sat + jnp.sum(rc * sat))

            s0, cnt_ok, sum_sat = lax.fori_loop(
                0, _NBINS // _L, chunk_body,
                (jnp.int32(0), jnp.int32(0), jnp.int32(0)))
            bstar = cnt_ok - jnp.int32(1)
            kr = kr - (s0 - sum_sat)
            pref = (pref << 8) | bstar
        return jnp.where(lanes == rl, pref, thr_vec)

    thrv[...] = lax.fori_loop(0, _RPW, row_body, jnp.zeros((_L,), jnp.int32))
    pltpu.sync_copy(thrv, thr_hbm.at[wid])


_sc_thresholds = functools.partial(
    pl.kernel,
    out_type=jax.ShapeDtypeStruct((_NW, _L), jnp.int32),
    mesh=plsc.VectorSubcoreMesh(core_axis_name="c", subcore_axis_name="s",
                                num_cores=_NC, num_subcores=_NS),
    compiler_params=pltpu.CompilerParams(needs_layout_passes=False),
    scratch_types=[
        pltpu.VMEM((_N,), jnp.float32),
        pltpu.VMEM((_N,), jnp.float32),
        pltpu.VMEM((_NBINS * _L,), jnp.int32),
        pltpu.VMEM((_L,), jnp.int32),
    ],
)(_sc_body)

_ROWS = 8


def _mask_body(x_ref, scale_ref, t_ref, out_ref):
    x = x_ref[...]
    u = _encode(lax.bitcast_convert_type(x * scale_ref[...], jnp.int32))
    uu = lax.bitcast_convert_type(u, jnp.uint32)
    tt = lax.bitcast_convert_type(t_ref[...], jnp.uint32)
    out_ref[...] = jnp.where(uu >= tt, x, jnp.float32(0.0))


def _tc_mask(x, scale, thr):
    return pl.pallas_call(
        _mask_body,
        grid=(_B // _ROWS,),
        in_specs=[
            pl.BlockSpec((_ROWS, _N), lambda i: (i, 0)),
            pl.BlockSpec((1, _N), lambda i: (0, 0)),
            pl.BlockSpec((_ROWS, 1), lambda i: (i, 0)),
        ],
        out_specs=pl.BlockSpec((_ROWS, _N), lambda i: (i, 0)),
        out_shape=jax.ShapeDtypeStruct((_B, _N), jnp.float32),
    )(x, scale, thr)


def kernel(x, duty_cycle):
    scale = jnp.exp(-_BOOST * duty_cycle)
    thr = _sc_thresholds(x, scale)
    t128 = thr[:, :_RPW].reshape(_B, 1)
    return _tc_mask(x, scale.reshape(1, _N), t128)


# speedup vs baseline: 1.2952x; 1.2952x over previous
import functools

import jax
import jax.numpy as jnp
from jax import lax
from jax.experimental import pallas as pl
from jax.experimental.pallas import tpu as pltpu
from jax.experimental.pallas import tpu_sc as plsc

_N = 32768
_B = 128
_K = 3277
_BOOST = 1.0

_NC, _NS, _L = 2, 16, 16
_NW = _NC * _NS
_RPW = _B // _NW
_NBINS = 256
_NLEV = 4
_UF = 8


def _encode(bits):
    m = lax.shift_right_arithmetic(bits, 31)
    return bits ^ (m | jnp.int32(-2147483648))


def _srl(v, amount):
    return lax.shift_right_logical(v, lax.full_like(v, amount))


def _sc_body(x_hbm, scale_hbm, thr_hbm, xrow, scalev, urow, hist, thrv):
    cid = lax.axis_index("c")
    sid = lax.axis_index("s")
    wid = sid * _NC + cid
    pltpu.sync_copy(scale_hbm, scalev)
    lanes = lax.iota(jnp.int32, _L)
    lane_base = lanes * _NBINS
    ones = jnp.ones((_L,), jnp.int32)

    def zero_hist():
        def zero_body(j, _):
            for jj in range(_UF):
                hist[pl.ds((j * _UF + jj) * _L, _L)] = jnp.zeros(
                    (_L,), jnp.int32)
            return 0

        lax.fori_loop(0, _NBINS // _UF, zero_body, 0)

    def pick_bin(kr):
        def chunk_body(j, carry):
            csum, cntv, satv = carry
            cbase = (_NBINS // _L - 1 - j) * _L
            counts = jnp.zeros((_L,), jnp.int32)
            for l in range(_L):
                counts = counts + hist[pl.ds(l * _NBINS + cbase, _L)]
            rc = lax.rev(counts, (0,))
            rs = jnp.cumsum(rc) + csum
            sat = (rs >= kr).astype(jnp.int32)
            return rs[_L - 1], cntv + sat, satv + rc * sat

        csum, cntv, satv = lax.fori_loop(
            0, _NBINS // _L, chunk_body,
            (jnp.int32(0), jnp.zeros((_L,), jnp.int32),
             jnp.zeros((_L,), jnp.int32)))
        return jnp.sum(cntv) - jnp.int32(1), csum, jnp.sum(satv)

    def row_body(rl, thr_vec):
        pltpu.sync_copy(x_hbm.at[wid * _RPW + rl], xrow)

        def lev0_body(i, _):
            for jj in range(_UF):
                ds = pl.ds((i * _UF + jj) * _L, _L)
                xv = xrow[ds]
                sv = scalev[ds]
                u = _encode(lax.bitcast_convert_type(xv * sv, jnp.int32))
                urow[ds] = u
                plsc.addupdate_scatter(hist, [lane_base + _srl(u, 24)], ones)
            return 0

        zero_hist()
        lax.fori_loop(0, _N // _L // _UF, lev0_body, 0)
        bstar, s0, sum_sat = pick_bin(jnp.int32(_K))
        pref = bstar
        kr = jnp.int32(_K) - (s0 - sum_sat)

        for lev in range(1, _NLEV):
            shift = 24 - 8 * lev

            def scan_body(i, _, shift=shift, pref=pref):
                for jj in range(_UF):
                    ds = pl.ds((i * _UF + jj) * _L, _L)
                    u = urow[ds]
                    binv = _srl(u, shift) & 255
                    match = _srl(u, shift + 8) == pref
                    plsc.addupdate_scatter(hist, [lane_base + binv], ones,
                                           mask=match)
                return 0

            zero_hist()
            lax.fori_loop(0, _N // _L // _UF, scan_body, 0)
            bstar, s0, sum_sat = pick_bin(kr)
            kr = kr - (s0 - sum_sat)
            pref = (pref << 8) | bstar
        return jnp.where(lanes == rl, pref, thr_vec)

    thrv[...] = lax.fori_loop(0, _RPW, row_body, jnp.zeros((_L,), jnp.int32))
    pltpu.sync_copy(thrv, thr_hbm.at[wid])


_sc_thresholds = functools.partial(
    pl.kernel,
    out_type=jax.ShapeDtypeStruct((_NW, _L), jnp.int32),
    mesh=plsc.VectorSubcoreMesh(core_axis_name="c", subcore_axis_name="s",
                                num_cores=_NC, num_subcores=_NS),
    compiler_params=pltpu.CompilerParams(needs_layout_passes=False),
    scratch_types=[
        pltpu.VMEM((_N,), jnp.float32),
        pltpu.VMEM((_N,), jnp.float32),
        pltpu.VMEM((_N,), jnp.int32),
        pltpu.VMEM((_NBINS * _L,), jnp.int32),
        pltpu.VMEM((_L,), jnp.int32),
    ],
)(_sc_body)

_ROWS = 8


def _mask_body(x_ref, scale_ref, t_ref, out_ref):
    x = x_ref[...]
    u = _encode(lax.bitcast_convert_type(x * scale_ref[...], jnp.int32))
    uu = lax.bitcast_convert_type(u, jnp.uint32)
    tt = lax.bitcast_convert_type(t_ref[...], jnp.uint32)
    out_ref[...] = jnp.where(uu >= tt, x, jnp.float32(0.0))


def _tc_mask(x, scale, thr):
    return pl.pallas_call(
        _mask_body,
        grid=(_B // _ROWS,),
        in_specs=[
            pl.BlockSpec((_ROWS, _N), lambda i: (i, 0)),
            pl.BlockSpec((1, _N), lambda i: (0, 0)),
            pl.BlockSpec((_ROWS, 1), lambda i: (i, 0)),
        ],
        out_specs=pl.BlockSpec((_ROWS, _N), lambda i: (i, 0)),
        out_shape=jax.ShapeDtypeStruct((_B, _N), jnp.float32),
    )(x, scale, thr)


def kernel(x, duty_cycle):
    scale = jnp.exp(-_BOOST * duty_cycle)
    thr = _sc_thresholds(x, scale)
    t128 = thr[:, :_RPW].reshape(_B, 1)
    return _tc_mask(x, scale.reshape(1, _N), t128)


# speedup vs baseline: 4.2716x; 3.2979x over previous
import functools

import jax
import jax.numpy as jnp
from jax import lax
from jax.experimental import pallas as pl
from jax.experimental.pallas import tpu as pltpu
from jax.experimental.pallas import tpu_sc as plsc

_N = 32768
_B = 128
_K = 3277
_BOOST = 1.0

_NC, _NS, _L = 2, 16, 16
_NW = _NC * _NS
_RPW = _B // _NW
_NBINS = 256
_NLEV = 4
_UF = 8


def _encode(bits):
    m = lax.shift_right_arithmetic(bits, 31)
    return bits ^ (m | jnp.int32(-2147483648))


def _srl(v, amount):
    return lax.shift_right_logical(v, lax.full_like(v, amount))


def _sc_body(x_hbm, scale_hbm, thr_hbm, xrow, scalev, urow, hist, thrv):
    cid = lax.axis_index("c")
    sid = lax.axis_index("s")
    wid = sid * _NC + cid
    pltpu.sync_copy(scale_hbm, scalev)
    lanes = lax.iota(jnp.int32, _L)
    ones = jnp.ones((_L,), jnp.int32)

    def zero_hist():
        @plsc.parallel_loop(0, _NBINS * _L, step=_L, unroll=_UF)
        def _(i):
            hist[pl.ds(i, _L)] = jnp.zeros((_L,), jnp.int32)

    def pick_bin(kr):
        sc = []
        for c in range(_L):
            tvec = hist[pl.ds(c * _NBINS, _L)]
            for k in range(1, _L):
                tvec = tvec + hist[pl.ds(c * _NBINS + k * _L, _L)]
            sc.append(jnp.sum(tvec))
        cums = [jnp.int32(0)] * (_L + 1)
        for c in range(_L - 1, -1, -1):
            cums[c] = cums[c + 1] + sc[c]
        c_star = sum(((cums[c] >= kr).astype(jnp.int32) for c in range(_L)),
                     jnp.int32(0)) - jnp.int32(1)
        above_c = sum((jnp.where(jnp.int32(c) > c_star, sc[c], 0)
                       for c in range(_L)), jnp.int32(0))
        kr1 = kr - above_c
        fj = [jnp.sum(hist[pl.ds(c_star * _NBINS + j * _L, _L)])
              for j in range(_L)]
        fsuf = [jnp.int32(0)] * (_L + 1)
        for j in range(_L - 1, -1, -1):
            fsuf[j] = fsuf[j + 1] + fj[j]
        j_star = sum(((fsuf[j] >= kr1).astype(jnp.int32) for j in range(_L)),
                     jnp.int32(0)) - jnp.int32(1)
        above_j = sum((jnp.where(jnp.int32(j) > j_star, fj[j], 0)
                       for j in range(_L)), jnp.int32(0))
        return c_star * _L + j_star, kr1 - above_j

    def row_body(rl, thr_vec):
        pltpu.sync_copy(x_hbm.at[wid * _RPW + rl], xrow)

        zero_hist()

        @plsc.parallel_loop(0, _N, step=_L, unroll=_UF)
        def _(i):
            ds = pl.ds(i, _L)
            u = _encode(lax.bitcast_convert_type(xrow[ds] * scalev[ds],
                                                 jnp.int32))
            urow[ds] = u
            plsc.addupdate_scatter(hist, [_srl(u, 24) * _L + lanes], ones)

        bstar, kr = pick_bin(jnp.int32(_K))
        pref = bstar

        for lev in range(1, _NLEV):
            shift = 24 - 8 * lev
            zero_hist()

            @plsc.parallel_loop(0, _N, step=_L, unroll=_UF)
            def _(i, shift=shift, pref=pref):
                ds = pl.ds(i, _L)
                u = urow[ds]
                binv = _srl(u, shift) & 255
                match = _srl(u, shift + 8) == pref
                plsc.addupdate_scatter(hist, [binv * _L + lanes], ones,
                                       mask=match)

            bstar, kr = pick_bin(kr)
            pref = (pref << 8) | bstar
        return jnp.where(lanes == rl, pref, thr_vec)

    thrv[...] = lax.fori_loop(0, _RPW, row_body, jnp.zeros((_L,), jnp.int32))
    pltpu.sync_copy(thrv, thr_hbm.at[wid])


_sc_thresholds = functools.partial(
    pl.kernel,
    out_type=jax.ShapeDtypeStruct((_NW, _L), jnp.int32),
    mesh=plsc.VectorSubcoreMesh(core_axis_name="c", subcore_axis_name="s",
                                num_cores=_NC, num_subcores=_NS),
    compiler_params=pltpu.CompilerParams(needs_layout_passes=False),
    scratch_types=[
        pltpu.VMEM((_N,), jnp.float32),
        pltpu.VMEM((_N,), jnp.float32),
        pltpu.VMEM((_N,), jnp.int32),
        pltpu.VMEM((_NBINS * _L,), jnp.int32),
        pltpu.VMEM((_L,), jnp.int32),
    ],
)(_sc_body)

_ROWS = 8


def _mask_body(x_ref, scale_ref, t_ref, out_ref):
    x = x_ref[...]
    u = _encode(lax.bitcast_convert_type(x * scale_ref[...], jnp.int32))
    uu = lax.bitcast_convert_type(u, jnp.uint32)
    tt = lax.bitcast_convert_type(t_ref[...], jnp.uint32)
    out_ref[...] = jnp.where(uu >= tt, x, jnp.float32(0.0))


def _tc_mask(x, scale, thr):
    return pl.pallas_call(
        _mask_body,
        grid=(_B // _ROWS,),
        in_specs=[
            pl.BlockSpec((_ROWS, _N), lambda i: (i, 0)),
            pl.BlockSpec((1, _N), lambda i: (0, 0)),
            pl.BlockSpec((_ROWS, 1), lambda i: (i, 0)),
        ],
        out_specs=pl.BlockSpec((_ROWS, _N), lambda i: (i, 0)),
        out_shape=jax.ShapeDtypeStruct((_B, _N), jnp.float32),
    )(x, scale, thr)


def kernel(x, duty_cycle):
    scale = jnp.exp(-_BOOST * duty_cycle)
    thr = _sc_thresholds(x, scale)
    t128 = thr[:, :_RPW].reshape(_B, 1)
    return _tc_mask(x, scale.reshape(1, _N), t128)


# speedup vs baseline: 4.5371x; 1.0621x over previous
import functools

import jax
import jax.numpy as jnp
from jax import lax
from jax.experimental import pallas as pl
from jax.experimental.pallas import tpu as pltpu
from jax.experimental.pallas import tpu_sc as plsc

_N = 32768
_B = 128
_K = 3277
_BOOST = 1.0

_NC, _NS, _L = 2, 16, 16
_NW = _NC * _NS
_RPW = _B // _NW
_NBINS = 256
_NLEV = 4
_UF = 8


def _encode(bits):
    m = lax.shift_right_arithmetic(bits, 31)
    return bits ^ (m | jnp.int32(-2147483648))


def _srl(v, amount):
    return lax.shift_right_logical(v, lax.full_like(v, amount))


def _sc_body(x_hbm, scale_hbm, out_hbm, xrow, scalev, urow, hist):
    cid = lax.axis_index("c")
    sid = lax.axis_index("s")
    wid = sid * _NC + cid
    pltpu.sync_copy(scale_hbm, scalev)
    lanes = lax.iota(jnp.int32, _L)
    ones = jnp.ones((_L,), jnp.int32)

    def zero_hist():
        @plsc.parallel_loop(0, _NBINS * _L, step=_L, unroll=_UF)
        def _(i):
            hist[pl.ds(i, _L)] = jnp.zeros((_L,), jnp.int32)

    def pick_bin(kr):
        sc = []
        for c in range(_L):
            tvec = hist[pl.ds(c * _NBINS, _L)]
            for k in range(1, _L):
                tvec = tvec + hist[pl.ds(c * _NBINS + k * _L, _L)]
            sc.append(jnp.sum(tvec))
        cums = [jnp.int32(0)] * (_L + 1)
        for c in range(_L - 1, -1, -1):
            cums[c] = cums[c + 1] + sc[c]
        c_star = sum(((cums[c] >= kr).astype(jnp.int32) for c in range(_L)),
                     jnp.int32(0)) - jnp.int32(1)
        above_c = sum((jnp.where(jnp.int32(c) > c_star, sc[c], 0)
                       for c in range(_L)), jnp.int32(0))
        kr1 = kr - above_c
        fj = [jnp.sum(hist[pl.ds(c_star * _NBINS + j * _L, _L)])
              for j in range(_L)]
        fsuf = [jnp.int32(0)] * (_L + 1)
        for j in range(_L - 1, -1, -1):
            fsuf[j] = fsuf[j + 1] + fj[j]
        j_star = sum(((fsuf[j] >= kr1).astype(jnp.int32) for j in range(_L)),
                     jnp.int32(0)) - jnp.int32(1)
        above_j = sum((jnp.where(jnp.int32(j) > j_star, fj[j], 0)
                       for j in range(_L)), jnp.int32(0))
        return c_star * _L + j_star, kr1 - above_j

    def row_body(rl, _):
        pltpu.sync_copy(x_hbm.at[wid * _RPW + rl], xrow)

        zero_hist()

        @plsc.parallel_loop(0, _N, step=_L, unroll=_UF)
        def _(i):
            ds = pl.ds(i, _L)
            u = _encode(lax.bitcast_convert_type(xrow[ds] * scalev[ds],
                                                 jnp.int32))
            urow[ds] = u
            plsc.addupdate_scatter(hist, [_srl(u, 24) * _L + lanes], ones)

        bstar, kr = pick_bin(jnp.int32(_K))
        pref = bstar

        for lev in range(1, _NLEV):
            shift = 24 - 8 * lev
            zero_hist()

            @plsc.parallel_loop(0, _N, step=_L, unroll=_UF)
            def _(i, shift=shift, pref=pref):
                ds = pl.ds(i, _L)
                u = urow[ds]
                binv = _srl(u, shift) & 255
                match = _srl(u, shift + 8) == pref
                plsc.addupdate_scatter(hist, [binv * _L + lanes], ones,
                                       mask=match)

            bstar, kr = pick_bin(kr)
            pref = (pref << 8) | bstar

        prefu = pref ^ jnp.int32(-2147483648)

        @plsc.parallel_loop(0, _N, step=_L, unroll=_UF)
        def _(i):
            ds = pl.ds(i, _L)
            keep = (urow[ds] ^ jnp.int32(-2147483648)) >= prefu
            xrow[ds] = jnp.where(keep, xrow[ds], jnp.float32(0.0))

        pltpu.sync_copy(xrow, out_hbm.at[wid * _RPW + rl])
        return 0

    lax.fori_loop(0, _RPW, row_body, 0)


_sc_kwinners = functools.partial(
    pl.kernel,
    out_type=jax.ShapeDtypeStruct((_B, _N), jnp.float32),
    mesh=plsc.VectorSubcoreMesh(core_axis_name="c", subcore_axis_name="s",
                                num_cores=_NC, num_subcores=_NS),
    compiler_params=pltpu.CompilerParams(needs_layout_passes=False),
    scratch_types=[
        pltpu.VMEM((_N,), jnp.float32),
        pltpu.VMEM((_N,), jnp.float32),
        pltpu.VMEM((_N,), jnp.int32),
        pltpu.VMEM((_NBINS * _L,), jnp.int32),
    ],
)(_sc_body)

_ROWS = 8


def _mask_body(x_ref, scale_ref, t_ref, out_ref):
    x = x_ref[...]
    u = _encode(lax.bitcast_convert_type(x * scale_ref[...], jnp.int32))
    uu = lax.bitcast_convert_type(u, jnp.uint32)
    tt = lax.bitcast_convert_type(t_ref[...], jnp.uint32)
    out_ref[...] = jnp.where(uu >= tt, x, jnp.float32(0.0))


def _tc_mask(x, scale, thr):
    return pl.pallas_call(
        _mask_body,
        grid=(_B // _ROWS,),
        in_specs=[
            pl.BlockSpec((_ROWS, _N), lambda i: (i, 0)),
            pl.BlockSpec((1, _N), lambda i: (0, 0)),
            pl.BlockSpec((_ROWS, 1), lambda i: (i, 0)),
        ],
        out_specs=pl.BlockSpec((_ROWS, _N), lambda i: (i, 0)),
        out_shape=jax.ShapeDtypeStruct((_B, _N), jnp.float32),
    )(x, scale, thr)


def kernel(x, duty_cycle):
    scale = jnp.exp(-_BOOST * duty_cycle)
    return _sc_kwinners(x, scale)
